# Initial kernel scaffold; baseline (speedup 1.0000x reference)
import jax
import jax.numpy as jnp
from jax.experimental import pallas as pl


def kernel(edge_embedding, edge_paths, edge_vector):
    raise NotImplementedError("write your pallas kernel here")


# speedup vs baseline: 67.2738x; 67.2738x over previous
import functools

import jax
import jax.numpy as jnp
from jax import lax
from jax.experimental import pallas as pl
from jax.experimental.pallas import tpu as pltpu
from jax.experimental.pallas import tpu_sc as plsc

B = 4
V = 1024
D = 16
N = 256
L = 5
LP = 8
NW = 32
NELEM = B * N * N
EPT = NELEM // NW
IPT = EPT * L
TABW = LP * V

_SCALE = 1.0 / (5.0 + 1e-9)


def _table_body(vec_ref, emb_ref, out_ref):
    emb = emb_ref[0]
    vec = vec_ref[...]
    out_ref[0] = lax.dot_general(
        vec, emb, (((1,), (1,)), ((), ())),
        preferred_element_type=jnp.float32) * _SCALE


def _make_table(vec8, emb):
    return pl.pallas_call(
        _table_body,
        grid=(B,),
        in_specs=[
            pl.BlockSpec((LP, D), lambda b: (0, 0)),
            pl.BlockSpec((1, V, D), lambda b: (b, 0, 0)),
        ],
        out_specs=pl.BlockSpec((1, LP, V), lambda b: (b, 0, 0)),
        out_shape=jax.ShapeDtypeStruct((B, LP, V), jnp.float32),
    )(vec8, emb)


@functools.partial(
    pl.kernel,
    out_type=jax.ShapeDtypeStruct((NELEM,), jnp.float32),
    mesh=plsc.VectorSubcoreMesh(core_axis_name="c", subcore_axis_name="s"),
    compiler_params=pltpu.CompilerParams(needs_layout_passes=False),
    scratch_types=[
        pltpu.VMEM((IPT,), jnp.int32),
        pltpu.VMEM((TABW,), jnp.float32),
        pltpu.VMEM((EPT,), jnp.float32),
    ],
)
def _sc_gather(idx_hbm, tab_hbm, out_hbm, idx_v, tab_v, out_v):
    wid = lax.axis_index("s") * 2 + lax.axis_index("c")
    b = wid // (NW // B)
    pltpu.sync_copy(idx_hbm.at[pl.ds(wid * IPT, IPT)], idx_v)
    pltpu.sync_copy(tab_hbm.at[pl.ds(b * TABW, TABW)], tab_v)
    iota5 = lax.iota(jnp.int32, 16) * L

    def body(i, carry):
        base = i * (16 * L)
        acc = jnp.zeros((16,), jnp.float32)
        for l in range(L):
            pos = iota5 + (base + l)
            ii = plsc.load_gather(idx_v, [pos])
            acc = acc + plsc.load_gather(tab_v, [ii + l * V])
        out_v[pl.ds(i * 16, 16)] = acc
        return carry

    lax.fori_loop(0, EPT // 16, body, 0, unroll=4)
    pltpu.sync_copy(out_v, out_hbm.at[pl.ds(wid * EPT, EPT)])


def kernel(edge_embedding, edge_paths, edge_vector):
    vec8 = jnp.zeros((LP, D), jnp.float32).at[:L].set(
        edge_vector.astype(jnp.float32))
    tab = _make_table(vec8, edge_embedding)
    idx_flat = edge_paths.reshape(-1).astype(jnp.int32)
    out = _sc_gather(idx_flat, tab.reshape(-1))
    return out.reshape(B, N, N)


# speedup vs baseline: 364.3916x; 5.4165x over previous
import functools

import jax
import jax.numpy as jnp
from jax import lax
from jax.experimental import pallas as pl
from jax.experimental.pallas import tpu as pltpu
from jax.experimental.pallas import tpu_sc as plsc

B = 4
V = 1024
D = 16
N = 256
L = 5
LP = 8
NW = 32
NELEM = B * N * N
PLANE = N * N
EPT = NELEM // NW
TPB = NW // B
TABW = LP * V

_SCALE = 1.0 / (5.0 + 1e-9)


def _table_body(vec_ref, emb_ref, out_ref):
    emb = emb_ref[0]
    vec = vec_ref[...]
    out_ref[0] = lax.dot_general(
        vec, emb, (((1,), (1,)), ((), ())),
        preferred_element_type=jnp.float32) * _SCALE


def _make_table(vec8, emb):
    return pl.pallas_call(
        _table_body,
        grid=(B,),
        in_specs=[
            pl.BlockSpec((LP, D), lambda b: (0, 0)),
            pl.BlockSpec((1, V, D), lambda b: (b, 0, 0)),
        ],
        out_specs=pl.BlockSpec((1, LP, V), lambda b: (b, 0, 0)),
        out_shape=jax.ShapeDtypeStruct((B, LP, V), jnp.float32),
    )(vec8, emb)


@functools.partial(
    pl.kernel,
    out_type=jax.ShapeDtypeStruct((NELEM,), jnp.float32),
    mesh=plsc.VectorSubcoreMesh(core_axis_name="c", subcore_axis_name="s"),
    compiler_params=pltpu.CompilerParams(needs_layout_passes=False),
    scratch_types=[
        pltpu.VMEM((L * EPT,), jnp.int32),
        pltpu.VMEM((TABW,), jnp.float32),
        pltpu.VMEM((EPT,), jnp.float32),
    ],
)
def _sc_gather(idx_hbm, tab_hbm, out_hbm, idx_v, tab_v, out_v):
    wid = lax.axis_index("s") * 2 + lax.axis_index("c")
    b = wid // TPB
    chunk = wid % TPB
    for l in range(L):
        pltpu.sync_copy(
            idx_hbm.at[pl.ds((b * L + l) * PLANE + chunk * EPT, EPT)],
            idx_v.at[pl.ds(l * EPT, EPT)])
    pltpu.sync_copy(tab_hbm.at[pl.ds(b * TABW, TABW)], tab_v)

    def body(i, carry):
        acc = jnp.zeros((16,), jnp.float32)
        for l in range(L):
            iv = idx_v[pl.ds(l * EPT + i * 16, 16)]
            acc = acc + plsc.load_gather(tab_v, [iv + l * V])
        out_v[pl.ds(i * 16, 16)] = acc
        return carry

    lax.fori_loop(0, EPT // 16, body, 0, unroll=4)
    pltpu.sync_copy(out_v, out_hbm.at[pl.ds(wid * EPT, EPT)])


def kernel(edge_embedding, edge_paths, edge_vector):
    vec8 = jnp.zeros((LP, D), jnp.float32).at[:L].set(
        edge_vector.astype(jnp.float32))
    tab = _make_table(vec8, edge_embedding)
    idx_t = (edge_paths.astype(jnp.int32)
             .reshape(B, 32, 8, 2, 128, L)
             .transpose(0, 5, 1, 3, 2, 4)
             .reshape(-1))
    out = _sc_gather(idx_t, tab.reshape(-1))
    return (out.reshape(B, 32, 2, 8, 128)
            .transpose(0, 1, 3, 2, 4)
            .reshape(B, N, N))


# speedup vs baseline: 492.0943x; 1.3505x over previous
import functools

import jax
import jax.numpy as jnp
from jax import lax
from jax.experimental import pallas as pl
from jax.experimental.pallas import tpu as pltpu
from jax.experimental.pallas import tpu_sc as plsc

B = 4
V = 1024
D = 16
N = 256
L = 5
NW = 32
NELEM = B * N * N
PLANE = N * N
EPT = NELEM // NW
TPB = NW // B
TABW = L * V

_SCALE = 1.0 / (5.0 + 1e-9)


def _table_body(vec_ref, emb_ref, out_ref):
    vec = vec_ref[...]
    emb = emb_ref[0]
    for j in range(V // 128):
        out_ref[0, :, j, :] = lax.dot_general(
            vec, emb[:, j * 128:(j + 1) * 128], (((1,), (0,)), ((), ())),
            preferred_element_type=jnp.float32) * _SCALE


def _make_table(vec, emb_t):
    return pl.pallas_call(
        _table_body,
        grid=(B,),
        in_specs=[
            pl.BlockSpec((L, D), lambda b: (0, 0)),
            pl.BlockSpec((1, D, V), lambda b: (b, 0, 0)),
        ],
        out_specs=pl.BlockSpec((1, L, V // 128, 128), lambda b: (b, 0, 0, 0)),
        out_shape=jax.ShapeDtypeStruct((B, L, V // 128, 128), jnp.float32),
    )(vec, emb_t)


@functools.partial(
    pl.kernel,
    out_type=jax.ShapeDtypeStruct((NELEM,), jnp.float32),
    mesh=plsc.VectorSubcoreMesh(core_axis_name="c", subcore_axis_name="s"),
    compiler_params=pltpu.CompilerParams(needs_layout_passes=False),
    scratch_types=[
        pltpu.VMEM((L * EPT,), jnp.int32),
        pltpu.VMEM((TABW,), jnp.float32),
        pltpu.VMEM((EPT,), jnp.float32),
        pltpu.SemaphoreType.DMA,
    ],
)
def _sc_gather(idx_hbm, tab_hbm, out_hbm, idx_v, tab_v, out_v, sem):
    wid = lax.axis_index("s") * 2 + lax.axis_index("c")
    b = wid // TPB
    chunk = wid % TPB
    copies = [
        pltpu.async_copy(
            idx_hbm.at[pl.ds((b * L + l) * PLANE + chunk * EPT, EPT)],
            idx_v.at[pl.ds(l * EPT, EPT)], sem)
        for l in range(L)
    ]
    copies.append(
        pltpu.async_copy(tab_hbm.at[pl.ds(b * TABW, TABW)], tab_v, sem))
    for c in copies:
        c.wait()

    def body(i, carry):
        acc = jnp.zeros((16,), jnp.float32)
        for l in range(L):
            iv = idx_v[pl.ds(l * EPT + i * 16, 16)]
            acc = acc + plsc.load_gather(tab_v.at[pl.ds(l * V, V)], [iv])
        out_v[pl.ds(i * 16, 16)] = acc
        return carry

    lax.fori_loop(0, EPT // 16, body, 0, unroll=8)
    pltpu.sync_copy(out_v, out_hbm.at[pl.ds(wid * EPT, EPT)])


def kernel(edge_embedding, edge_paths, edge_vector):
    emb_t = edge_embedding.transpose(0, 2, 1)
    tab = _make_table(edge_vector.astype(jnp.float32), emb_t)
    idx_t = (edge_paths.astype(jnp.int32)
             .reshape(B, 32, 8, 2, 128, L)
             .transpose(0, 5, 1, 3, 2, 4)
             .reshape(-1))
    out = _sc_gather(idx_t, tab.reshape(-1))
    return (out.reshape(B, 32, 2, 8, 128)
            .transpose(0, 1, 3, 2, 4)
            .reshape(B, N, N))
